# baseline (speedup 1.0000x reference)
import jax
import jax.numpy as jnp
from jax import lax
from jax.experimental import pallas as pl
from jax.experimental.pallas import tpu as pltpu
from jax.experimental.pallas import tpu_sc as plsc

B = 1024
L = 200
H = 64
OUT = 100000

NC = 2
NS = 16
NW = NC * NS
LANE = 16
B_PER_W = B // NW
NG = H // LANE

VROWS = 500000
D2 = 2 * H

CHUNK_A = 128
CHUNK_B = L - CHUNK_A


def _pool_body(ids2_hbm, off_hbm, emb2_hbm, out_hbm, idx_all, off_all,
               rows_a, rows_b, pooled, sem):
    wid = lax.axis_index("s") * NC + lax.axis_index("c")
    base = wid * B_PER_W
    base2 = pl.multiple_of(base * 2, 8)
    ibase = pl.multiple_of(base * L, 8)
    pltpu.sync_copy(ids2_hbm.at[pl.ds(base2, 2 * B_PER_W)], idx_all)
    pltpu.sync_copy(off_hbm.at[pl.ds(ibase, B_PER_W * L)], off_all)

    lanes = lax.iota(jnp.int32, LANE)
    cg = tuple(lanes + g * LANE for g in range(NG))
    zero = jnp.zeros((LANE,), jnp.float32)

    def accum(buf, rb, n, accs):
        rbs = jnp.full((LANE,), rb, jnp.int32)

        def tok(t, accs):
            ts = jnp.full((LANE,), t, jnp.int32)
            ov = plsc.load_gather(off_all, [ts + rbs])
            vals = tuple(plsc.load_gather(buf, [ts, ov + cg[g]])
                         for g in range(NG))
            return tuple(accs[g] + vals[g] for g in range(NG))

        return lax.fori_loop(0, n, tok, accs, unroll=4)

    for r in range(B_PER_W):
        c1 = pltpu.async_copy(emb2_hbm.at[idx_all.at[2 * r]], rows_a, sem)
        c2 = pltpu.async_copy(emb2_hbm.at[idx_all.at[2 * r + 1]], rows_b, sem)
        c1.wait()
        c2.wait()
        accs = tuple(jnp.zeros((LANE,), jnp.float32) for _ in range(NG))
        accs = accum(rows_a, r * L, CHUNK_A, accs)
        accs = accum(rows_b, r * L + CHUNK_A, CHUNK_B, accs)
        for g in range(NG):
            pooled[r, pl.ds(g * LANE, LANE)] = accs[g]
            pooled[r, pl.ds(H + g * LANE, LANE)] = zero

    pltpu.sync_copy(pooled, out_hbm.at[pl.ds(base, B_PER_W)])


def _make_pool():
    mesh = plsc.VectorSubcoreMesh(core_axis_name="c", subcore_axis_name="s",
                                  num_cores=NC, num_subcores=NS)
    return pl.kernel(
        _pool_body,
        out_type=jax.ShapeDtypeStruct((B, D2), jnp.float32),
        mesh=mesh,
        scratch_types=[
            pltpu.VMEM((2 * B_PER_W, CHUNK_A), jnp.int32),
            pltpu.VMEM((B_PER_W * L,), jnp.int32),
            pltpu.VMEM((CHUNK_A, D2), jnp.float32),
            pltpu.VMEM((CHUNK_A, D2), jnp.float32),
            pltpu.VMEM((B_PER_W, D2), jnp.float32),
            pltpu.SemaphoreType.DMA,
        ],
        compiler_params=pltpu.CompilerParams(needs_layout_passes=False),
    )


BO = 1024


def _mm_body(x_ref, wt_ref, b_ref, o_ref):
    x = jnp.maximum(x_ref[...][:, :H] * (1.0 / L), 0.0)
    o_ref[...] = lax.dot_general(
        wt_ref[...], x, (((0,), (1,)), ((), ())),
        preferred_element_type=jnp.float32) + b_ref[...]


def _make_mm():
    grid = (pl.cdiv(OUT, BO),)
    return pl.pallas_call(
        _mm_body,
        grid=grid,
        in_specs=[
            pl.BlockSpec((B, D2), lambda i: (0, 0)),
            pl.BlockSpec((H, BO), lambda i: (0, i)),
            pl.BlockSpec((BO, 1), lambda i: (i, 0)),
        ],
        out_specs=pl.BlockSpec((BO, B), lambda i: (i, 0)),
        out_shape=jax.ShapeDtypeStruct((OUT, B), jnp.float32),
    )


@jax.jit
def kernel(input_ids, token_type_ids, attention_mask, emb, W, b):
    ids32 = input_ids.astype(jnp.int32)
    idsh = ids32 >> 1
    off = ((ids32 & 1) << 6).reshape(-1)
    pad = jnp.zeros((B, CHUNK_A - CHUNK_B), jnp.int32)
    ids2 = jnp.concatenate(
        [idsh[:, :CHUNK_A], idsh[:, CHUNK_A:], pad], axis=1)
    ids2 = ids2.reshape(2 * B, CHUNK_A)
    emb2 = emb.reshape(VROWS, D2)
    pooled = _make_pool()(ids2, off, emb2)
    logits_t = _make_mm()(pooled, W.T, b.reshape(OUT, 1))
    return logits_t.T


# speedup vs baseline: 1.0007x; 1.0007x over previous
import jax
import jax.numpy as jnp
from jax import lax
from jax.experimental import pallas as pl
from jax.experimental.pallas import tpu as pltpu
from jax.experimental.pallas import tpu_sc as plsc

B = 1024
L = 200
H = 64
OUT = 100000

NC = 2
NS = 16
NW = NC * NS
LANE = 16
B_PER_W = B // NW
NG = H // LANE

VROWS = 500000
D2 = 2 * H

CHUNK_A = 128
CHUNK_B = L - CHUNK_A


def _pool_body(ids2_hbm, nev_hbm, emb2_hbm, out_hbm, idx_all, nev_v,
               rows_a, rows_b, pooled, sem):
    wid = lax.axis_index("s") * NC + lax.axis_index("c")
    base = wid * B_PER_W
    base2 = pl.multiple_of(base * 2, 8)
    pltpu.sync_copy(ids2_hbm.at[pl.ds(base2, 2 * B_PER_W)], idx_all)
    pltpu.sync_copy(nev_hbm.at[pl.ds(base, B_PER_W)], nev_v)

    zero = jnp.zeros((LANE,), jnp.float32)
    lanes = lax.iota(jnp.int32, LANE)

    def accum(buf, lo, hi, off, accs):
        def tok(t, accs):
            return tuple(accs[g] + buf[t, pl.ds(off + g * LANE, LANE)]
                         for g in range(NG))

        return lax.fori_loop(lo, hi, tok, accs)

    for r in range(B_PER_W):
        c1 = pltpu.async_copy(emb2_hbm.at[idx_all.at[2 * r]], rows_a, sem)
        c2 = pltpu.async_copy(emb2_hbm.at[idx_all.at[2 * r + 1]], rows_b, sem)
        c1.wait()
        c2.wait()
        nv = nev_v[pl.ds((r // LANE) * LANE, LANE)]
        ne = jnp.max(jnp.where(lanes == (r % LANE), nv, jnp.int32(-1)))
        ne_a = jnp.minimum(ne, CHUNK_A)
        ne_b = jnp.maximum(ne - CHUNK_A, 0)
        accs = tuple(jnp.zeros((LANE,), jnp.float32) for _ in range(NG))
        accs = accum(rows_a, 0, ne_a, 0, accs)
        accs = accum(rows_a, ne_a, CHUNK_A, H, accs)
        accs = accum(rows_b, 0, ne_b, 0, accs)
        accs = accum(rows_b, ne_b, CHUNK_B, H, accs)
        for g in range(NG):
            pooled[r, pl.ds(g * LANE, LANE)] = accs[g]
            pooled[r, pl.ds(H + g * LANE, LANE)] = zero

    pltpu.sync_copy(pooled, out_hbm.at[pl.ds(base, B_PER_W)])


def _make_pool():
    mesh = plsc.VectorSubcoreMesh(core_axis_name="c", subcore_axis_name="s",
                                  num_cores=NC, num_subcores=NS)
    return pl.kernel(
        _pool_body,
        out_type=jax.ShapeDtypeStruct((B, D2), jnp.float32),
        mesh=mesh,
        scratch_types=[
            pltpu.VMEM((2 * B_PER_W, CHUNK_A), jnp.int32),
            pltpu.VMEM((B_PER_W,), jnp.int32),
            pltpu.VMEM((CHUNK_A, D2), jnp.float32),
            pltpu.VMEM((CHUNK_A, D2), jnp.float32),
            pltpu.VMEM((B_PER_W, D2), jnp.float32),
            pltpu.SemaphoreType.DMA,
        ],
        compiler_params=pltpu.CompilerParams(needs_layout_passes=False),
    )


BO = 1024


def _mm_body(x_ref, wt_ref, b_ref, o_ref):
    x = jnp.maximum(x_ref[...][:, :H] * (1.0 / L), 0.0)
    o_ref[...] = lax.dot_general(
        wt_ref[...], x, (((0,), (1,)), ((), ())),
        preferred_element_type=jnp.float32) + b_ref[...]


def _make_mm():
    grid = (pl.cdiv(OUT, BO),)
    return pl.pallas_call(
        _mm_body,
        grid=grid,
        in_specs=[
            pl.BlockSpec((B, D2), lambda i: (0, 0)),
            pl.BlockSpec((H, BO), lambda i: (0, i)),
            pl.BlockSpec((BO, 1), lambda i: (i, 0)),
        ],
        out_specs=pl.BlockSpec((BO, B), lambda i: (i, 0)),
        out_shape=jax.ShapeDtypeStruct((OUT, B), jnp.float32),
    )


@jax.jit
def kernel(input_ids, token_type_ids, attention_mask, emb, W, b):
    ids32 = input_ids.astype(jnp.int32)
    key = ((ids32 & 1) << 19) | (ids32 >> 1)
    key = jnp.sort(key, axis=1)
    idsh = key & ((1 << 19) - 1)
    nev = jnp.sum((key >> 19) == 0, axis=1).astype(jnp.int32)
    pad = jnp.zeros((B, CHUNK_A - CHUNK_B), jnp.int32)
    ids2 = jnp.concatenate(
        [idsh[:, :CHUNK_A], idsh[:, CHUNK_A:], pad], axis=1)
    ids2 = ids2.reshape(2 * B, CHUNK_A)
    emb2 = emb.reshape(VROWS, D2)
    pooled = _make_pool()(ids2, nev, emb2)
    logits_t = _make_mm()(pooled, W.T, b.reshape(OUT, 1))
    return logits_t.T


# speedup vs baseline: 1.0015x; 1.0008x over previous
import jax
import jax.numpy as jnp
from jax import lax
from jax.experimental import pallas as pl
from jax.experimental.pallas import tpu as pltpu
from jax.experimental.pallas import tpu_sc as plsc

B = 1024
L = 200
H = 64
OUT = 100000

NC = 2
NS = 16
NW = NC * NS
LANE = 16
B_PER_W = B // NW
NG = H // LANE

VROWS = 500000
D2 = 2 * H

CHUNK_A = 128
CHUNK_B = L - CHUNK_A


def _pool_body(ids2_hbm, nev_hbm, emb2_hbm, out_hbm, idx_all, nev_v,
               rows_a, rows_b, pooled, sem):
    wid = lax.axis_index("s") * NC + lax.axis_index("c")
    base = wid * B_PER_W
    base2 = pl.multiple_of(base * 2, 8)
    pltpu.sync_copy(ids2_hbm.at[pl.ds(base2, 2 * B_PER_W)], idx_all)
    pltpu.sync_copy(nev_hbm.at[pl.ds(base, B_PER_W)], nev_v)

    zero = jnp.zeros((LANE,), jnp.float32)
    lanes = lax.iota(jnp.int32, LANE)

    def accum(buf, lo, hi, off, accs):
        def tok(t, accs):
            return tuple(accs[g] + buf[t, pl.ds(off + g * LANE, LANE)]
                         for g in range(NG))

        return lax.fori_loop(lo, hi, tok, accs)

    for r in range(B_PER_W):
        c1 = pltpu.async_copy(emb2_hbm.at[idx_all.at[2 * r]], rows_a, sem)
        c2 = pltpu.async_copy(emb2_hbm.at[idx_all.at[2 * r + 1]], rows_b, sem)
        c1.wait()
        c2.wait()
        nv = nev_v[pl.ds((r // LANE) * LANE, LANE)]
        ne = jnp.max(jnp.where(lanes == (r % LANE), nv, jnp.int32(-1)))
        ne_a = jnp.minimum(ne, CHUNK_A)
        ne_b = jnp.maximum(ne - CHUNK_A, 0)
        accs = tuple(jnp.zeros((LANE,), jnp.float32) for _ in range(NG))
        accs = accum(rows_a, 0, ne_a, 0, accs)
        accs = accum(rows_a, ne_a, CHUNK_A, H, accs)
        accs = accum(rows_b, 0, ne_b, 0, accs)
        accs = accum(rows_b, ne_b, CHUNK_B, H, accs)
        for g in range(NG):
            pooled[r, pl.ds(g * LANE, LANE)] = accs[g]
            pooled[r, pl.ds(H + g * LANE, LANE)] = zero

    pltpu.sync_copy(pooled, out_hbm.at[pl.ds(base, B_PER_W)])


def _make_pool():
    mesh = plsc.VectorSubcoreMesh(core_axis_name="c", subcore_axis_name="s",
                                  num_cores=NC, num_subcores=NS)
    return pl.kernel(
        _pool_body,
        out_type=jax.ShapeDtypeStruct((B, D2), jnp.float32),
        mesh=mesh,
        scratch_types=[
            pltpu.VMEM((2 * B_PER_W, CHUNK_A), jnp.int32),
            pltpu.VMEM((B_PER_W,), jnp.int32),
            pltpu.VMEM((CHUNK_A, D2), jnp.float32),
            pltpu.VMEM((CHUNK_A, D2), jnp.float32),
            pltpu.VMEM((B_PER_W, D2), jnp.float32),
            pltpu.SemaphoreType.DMA,
        ],
        compiler_params=pltpu.CompilerParams(use_tc_tiling_on_sc=False,
                                             needs_layout_passes=False),
    )


BO = 1024


def _mm_body(x_ref, wt_ref, b_ref, o_ref):
    x = jnp.maximum(x_ref[...][:, :H] * (1.0 / L), 0.0)
    o_ref[...] = lax.dot_general(
        wt_ref[...], x, (((0,), (1,)), ((), ())),
        preferred_element_type=jnp.float32) + b_ref[...]


def _make_mm():
    grid = (pl.cdiv(OUT, BO),)
    return pl.pallas_call(
        _mm_body,
        grid=grid,
        in_specs=[
            pl.BlockSpec((B, D2), lambda i: (0, 0)),
            pl.BlockSpec((H, BO), lambda i: (0, i)),
            pl.BlockSpec((BO, 1), lambda i: (i, 0)),
        ],
        out_specs=pl.BlockSpec((BO, B), lambda i: (i, 0)),
        out_shape=jax.ShapeDtypeStruct((OUT, B), jnp.float32),
    )


@jax.jit
def kernel(input_ids, token_type_ids, attention_mask, emb, W, b):
    ids32 = input_ids.astype(jnp.int32)
    key = ((ids32 & 1) << 19) | (ids32 >> 1)
    key = jnp.sort(key, axis=1)
    idsh = key & ((1 << 19) - 1)
    nev = jnp.sum((key >> 19) == 0, axis=1).astype(jnp.int32)
    pad = jnp.zeros((B, CHUNK_A - CHUNK_B), jnp.int32)
    ids2 = jnp.concatenate(
        [idsh[:, :CHUNK_A], idsh[:, CHUNK_A:], pad], axis=1)
    ids2 = ids2.reshape(2 * B, CHUNK_A)
    emb2 = emb.reshape(VROWS, D2)
    pooled = _make_pool()(ids2, nev, emb2)
    logits_t = _make_mm()(pooled, W.T, b.reshape(OUT, 1))
    return logits_t.T


# speedup vs baseline: 3.6345x; 3.6291x over previous
import jax
import jax.numpy as jnp
from jax import lax
from jax.experimental import pallas as pl
from jax.experimental.pallas import tpu as pltpu
from jax.experimental.pallas import tpu_sc as plsc

B = 1024
L = 200
H = 64
OUT = 100000

NC = 2
NS = 16
NW = NC * NS
LANE = 16
B_PER_W = B // NW
NG = H // LANE

VROWS = 500000
D2 = 2 * H

CHUNK_A = 128
CHUNK_B = L - CHUNK_A


def _pool_body(ids2_hbm, nev_hbm, emb2_hbm, out_hbm, idx_all, nev_v,
               rows_a0, rows_b0, rows_a1, rows_b1, pooled, sem0, sem1):
    wid = lax.axis_index("s") * NC + lax.axis_index("c")
    base = wid * B_PER_W
    base2 = pl.multiple_of(base * 2, 8)
    pltpu.sync_copy(ids2_hbm.at[pl.ds(base2, 2 * B_PER_W)], idx_all)
    pltpu.sync_copy(nev_hbm.at[pl.ds(base, B_PER_W)], nev_v)

    zero = jnp.zeros((LANE,), jnp.float32)
    lanes = lax.iota(jnp.int32, LANE)
    bufs = ((rows_a0, rows_b0), (rows_a1, rows_b1))
    sems = (sem0, sem1)

    def start(r):
        ba, bb = bufs[r & 1]
        sem = sems[r & 1]
        c1 = pltpu.async_copy(emb2_hbm.at[idx_all.at[2 * r]], ba, sem)
        c2 = pltpu.async_copy(emb2_hbm.at[idx_all.at[2 * r + 1]], bb, sem)
        return (c1, c2)

    def accum(buf, lo, hi, off, accs):
        def tok(t, accs):
            return tuple(accs[g] + buf[t, pl.ds(off + g * LANE, LANE)]
                         for g in range(NG))

        return lax.fori_loop(lo, hi, tok, accs)

    cps = start(0)
    for r in range(B_PER_W):
        nxt = start(r + 1) if r + 1 < B_PER_W else None
        for c in cps:
            c.wait()
        rows_a, rows_b = bufs[r & 1]
        nv = nev_v[pl.ds((r // LANE) * LANE, LANE)]
        ne = jnp.max(jnp.where(lanes == (r % LANE), nv, jnp.int32(-1)))
        ne_a = jnp.minimum(ne, CHUNK_A)
        ne_b = jnp.maximum(ne - CHUNK_A, 0)
        accs = tuple(jnp.zeros((LANE,), jnp.float32) for _ in range(NG))
        accs = accum(rows_a, 0, ne_a, 0, accs)
        accs = accum(rows_a, ne_a, CHUNK_A, H, accs)
        accs = accum(rows_b, 0, ne_b, 0, accs)
        accs = accum(rows_b, ne_b, CHUNK_B, H, accs)
        for g in range(NG):
            pooled[r, pl.ds(g * LANE, LANE)] = accs[g]
            pooled[r, pl.ds(H + g * LANE, LANE)] = zero
        cps = nxt

    pltpu.sync_copy(pooled, out_hbm.at[pl.ds(base, B_PER_W)])


def _make_pool():
    mesh = plsc.VectorSubcoreMesh(core_axis_name="c", subcore_axis_name="s",
                                  num_cores=NC, num_subcores=NS)
    return pl.kernel(
        _pool_body,
        out_type=jax.ShapeDtypeStruct((B, D2), jnp.float32),
        mesh=mesh,
        scratch_types=[
            pltpu.VMEM((2 * B_PER_W, CHUNK_A), jnp.int32),
            pltpu.VMEM((B_PER_W,), jnp.int32),
            pltpu.VMEM((CHUNK_A, D2), jnp.float32),
            pltpu.VMEM((CHUNK_A, D2), jnp.float32),
            pltpu.VMEM((CHUNK_A, D2), jnp.float32),
            pltpu.VMEM((CHUNK_A, D2), jnp.float32),
            pltpu.VMEM((B_PER_W, D2), jnp.float32),
            pltpu.SemaphoreType.DMA,
            pltpu.SemaphoreType.DMA,
        ],
        compiler_params=pltpu.CompilerParams(use_tc_tiling_on_sc=False,
                                             needs_layout_passes=False),
    )


BO = 2048


def _mm_body(x_ref, wt_ref, b_ref, o_ref):
    x = jnp.maximum(x_ref[...][:, :H] * (1.0 / L), 0.0)
    o_ref[...] = lax.dot_general(
        wt_ref[...], x, (((0,), (1,)), ((), ())),
        preferred_element_type=jnp.float32) + b_ref[...]


def _make_mm():
    grid = (pl.cdiv(OUT, BO),)
    return pl.pallas_call(
        _mm_body,
        grid=grid,
        in_specs=[
            pl.BlockSpec((B, D2), lambda i: (0, 0)),
            pl.BlockSpec((H, BO), lambda i: (0, i)),
            pl.BlockSpec((BO, 1), lambda i: (i, 0)),
        ],
        out_specs=pl.BlockSpec((BO, B), lambda i: (i, 0)),
        out_shape=jax.ShapeDtypeStruct((OUT, B), jnp.float32),
    )


@jax.jit
def kernel(input_ids, token_type_ids, attention_mask, emb, W, b):
    ids32 = input_ids.astype(jnp.int32)
    key = ((ids32 & 1) << 19) | (ids32 >> 1)
    key = jnp.sort(key, axis=1)
    idsh = key & ((1 << 19) - 1)
    nev = jnp.sum((key >> 19) == 0, axis=1).astype(jnp.int32)
    pad = (jnp.arange(B, dtype=jnp.int32)[:, None] * 61
           + jnp.arange(CHUNK_A - CHUNK_B, dtype=jnp.int32)[None, :] * 977
           ) % VROWS
    ids2 = jnp.concatenate(
        [idsh[:, :CHUNK_A], idsh[:, CHUNK_A:], pad], axis=1)
    ids2 = ids2.reshape(2 * B, CHUNK_A)
    emb2 = emb.reshape(VROWS, D2)
    pooled = _make_pool()(ids2, nev, emb2)
    logits_t = _make_mm()(pooled, W.T, b.reshape(OUT, 1))
    return logits_t.T


# speedup vs baseline: 3.7474x; 1.0311x over previous
import jax
import jax.numpy as jnp
from jax import lax
from jax.experimental import pallas as pl
from jax.experimental.pallas import tpu as pltpu
from jax.experimental.pallas import tpu_sc as plsc

B = 1024
L = 200
H = 64
OUT = 100000

NC = 2
NS = 16
NW = NC * NS
LANE = 16
B_PER_W = B // NW
NG = H // LANE

VROWS = 500000
D2 = 2 * H

CHUNK_A = 128
CHUNK_B = L - CHUNK_A


def _pool_body(ids2_hbm, nev_hbm, emb2_hbm, out_hbm, idx_all, nev_v,
               rows_a0, rows_b0, rows_a1, rows_b1, pooled, sem0, sem1):
    wid = lax.axis_index("s") * NC + lax.axis_index("c")
    base = wid * B_PER_W
    base2 = pl.multiple_of(base * 2, 8)
    pltpu.sync_copy(ids2_hbm.at[pl.ds(base2, 2 * B_PER_W)], idx_all)
    pltpu.sync_copy(nev_hbm.at[pl.ds(base, B_PER_W)], nev_v)

    zero = jnp.zeros((LANE,), jnp.float32)
    lanes = lax.iota(jnp.int32, LANE)
    bufs = ((rows_a0, rows_b0), (rows_a1, rows_b1))
    sems = (sem0, sem1)

    def start(r):
        ba, bb = bufs[r & 1]
        sem = sems[r & 1]
        c1 = pltpu.async_copy(emb2_hbm.at[idx_all.at[2 * r]], ba, sem)
        c2 = pltpu.async_copy(emb2_hbm.at[idx_all.at[2 * r + 1]], bb, sem)
        return (c1, c2)

    def accum(buf, lo, hi, off, accs):
        def tok(t, accs):
            return tuple(accs[g] + buf[t, pl.ds(off + g * LANE, LANE)]
                         for g in range(NG))

        return lax.fori_loop(lo, hi, tok, accs)

    cps = start(0)
    for r in range(B_PER_W):
        nxt = start(r + 1) if r + 1 < B_PER_W else None
        for c in cps:
            c.wait()
        rows_a, rows_b = bufs[r & 1]
        nv = nev_v[pl.ds((r // LANE) * LANE, LANE)]
        ne = jnp.max(jnp.where(lanes == (r % LANE), nv, jnp.int32(-1)))
        ne_a = jnp.minimum(ne, CHUNK_A)
        ne_b = jnp.maximum(ne - CHUNK_A, 0)
        accs = tuple(jnp.zeros((LANE,), jnp.float32) for _ in range(NG))
        accs = accum(rows_a, 0, ne_a, 0, accs)
        accs = accum(rows_a, ne_a, CHUNK_A, H, accs)
        accs = accum(rows_b, 0, ne_b, 0, accs)
        accs = accum(rows_b, ne_b, CHUNK_B, H, accs)
        for g in range(NG):
            pooled[r, pl.ds(g * LANE, LANE)] = accs[g]
            pooled[r, pl.ds(H + g * LANE, LANE)] = zero
        cps = nxt

    pltpu.sync_copy(pooled, out_hbm.at[pl.ds(base, B_PER_W)])


def _make_pool():
    mesh = plsc.VectorSubcoreMesh(core_axis_name="c", subcore_axis_name="s",
                                  num_cores=NC, num_subcores=NS)
    return pl.kernel(
        _pool_body,
        out_type=jax.ShapeDtypeStruct((B, D2), jnp.float32),
        mesh=mesh,
        scratch_types=[
            pltpu.VMEM((2 * B_PER_W, CHUNK_A), jnp.int32),
            pltpu.VMEM((B_PER_W,), jnp.int32),
            pltpu.VMEM((CHUNK_A, D2), jnp.float32),
            pltpu.VMEM((CHUNK_A, D2), jnp.float32),
            pltpu.VMEM((CHUNK_A, D2), jnp.float32),
            pltpu.VMEM((CHUNK_A, D2), jnp.float32),
            pltpu.VMEM((B_PER_W, D2), jnp.float32),
            pltpu.SemaphoreType.DMA,
            pltpu.SemaphoreType.DMA,
        ],
        compiler_params=pltpu.CompilerParams(needs_layout_passes=False),
    )


BO = 2048


def _mm_body(x_ref, wt_ref, b_ref, o_ref):
    x = jnp.maximum(x_ref[...][:, :H] * (1.0 / L), 0.0)
    bcol = b_ref[...].reshape(BO, 1)
    o_ref[...] = lax.dot_general(
        wt_ref[...], x, (((0,), (1,)), ((), ())),
        preferred_element_type=jnp.float32) + bcol


def _make_mm():
    grid = (pl.cdiv(OUT, BO),)
    return pl.pallas_call(
        _mm_body,
        grid=grid,
        in_specs=[
            pl.BlockSpec((B, D2), lambda i: (0, 0)),
            pl.BlockSpec((H, BO), lambda i: (0, i)),
            pl.BlockSpec((1, BO), lambda i: (0, i)),
        ],
        out_specs=pl.BlockSpec((BO, B), lambda i: (i, 0)),
        out_shape=jax.ShapeDtypeStruct((OUT, B), jnp.float32),
    )


@jax.jit
def kernel(input_ids, token_type_ids, attention_mask, emb, W, b):
    ids32 = input_ids.astype(jnp.int32)
    key = ((ids32 & 1) << 19) | (ids32 >> 1)
    key = jnp.sort(key, axis=1)
    idsh = key & ((1 << 19) - 1)
    nev = jnp.sum((key >> 19) == 0, axis=1).astype(jnp.int32)
    pad = (jnp.arange(B, dtype=jnp.int32)[:, None] * 61
           + jnp.arange(CHUNK_A - CHUNK_B, dtype=jnp.int32)[None, :] * 977
           ) % VROWS
    ids2 = jnp.concatenate(
        [idsh[:, :CHUNK_A], idsh[:, CHUNK_A:], pad], axis=1)
    ids2 = ids2.reshape(2 * B, CHUNK_A)
    emb2 = emb.reshape(VROWS, D2)
    pooled = _make_pool()(ids2, nev, emb2)
    logits_t = _make_mm()(pooled, W.T, b.reshape(1, OUT))
    return logits_t.T


# speedup vs baseline: 3.7481x; 1.0002x over previous
import jax
import jax.numpy as jnp
from jax import lax
from jax.experimental import pallas as pl
from jax.experimental.pallas import tpu as pltpu
from jax.experimental.pallas import tpu_sc as plsc

B = 1024
L = 200
H = 64
OUT = 100000

NC = 2
NS = 16
NW = NC * NS
LANE = 16
B_PER_W = B // NW
NG = H // LANE

VROWS = 500000
D2 = 2 * H

CHUNK_A = 128
CHUNK_B = L - CHUNK_A


def _pool_body(ids2_hbm, nev_hbm, emb2_hbm, out_hbm, idx_all, nev_v,
               rows_a0, rows_b0, rows_a1, rows_b1, pooled, sem0, sem1):
    wid = lax.axis_index("s") * NC + lax.axis_index("c")
    base = wid * B_PER_W
    base2 = pl.multiple_of(base * 2, 8)
    pltpu.sync_copy(ids2_hbm.at[pl.ds(base2, 2 * B_PER_W)], idx_all)
    pltpu.sync_copy(nev_hbm.at[pl.ds(base, B_PER_W)], nev_v)

    zero = jnp.zeros((LANE,), jnp.float32)
    lanes = lax.iota(jnp.int32, LANE)
    bufs = ((rows_a0, rows_b0), (rows_a1, rows_b1))
    sems = (sem0, sem1)

    def start(r):
        ba, bb = bufs[r & 1]
        sem = sems[r & 1]
        c1 = pltpu.async_copy(emb2_hbm.at[idx_all.at[2 * r]], ba, sem)
        c2 = pltpu.async_copy(emb2_hbm.at[idx_all.at[2 * r + 1]], bb, sem)
        return (c1, c2)

    def accum(buf, lo, hi, off, accs):
        def tok(t, accs):
            return tuple(accs[g] + buf[t, pl.ds(off + g * LANE, LANE)]
                         for g in range(NG))

        return lax.fori_loop(lo, hi, tok, accs)

    cps = start(0)
    for r in range(B_PER_W):
        nxt = start(r + 1) if r + 1 < B_PER_W else None
        for c in cps:
            c.wait()
        rows_a, rows_b = bufs[r & 1]
        nv = nev_v[pl.ds((r // LANE) * LANE, LANE)]
        ne = jnp.max(jnp.where(lanes == (r % LANE), nv, jnp.int32(-1)))
        ne_a = jnp.minimum(ne, CHUNK_A)
        ne_b = jnp.maximum(ne - CHUNK_A, 0)
        accs = tuple(jnp.zeros((LANE,), jnp.float32) for _ in range(NG))
        accs = accum(rows_a, 0, ne_a, 0, accs)
        accs = accum(rows_a, ne_a, CHUNK_A, H, accs)
        accs = accum(rows_b, 0, ne_b, 0, accs)
        accs = accum(rows_b, ne_b, CHUNK_B, H, accs)
        for g in range(NG):
            pooled[r, pl.ds(g * LANE, LANE)] = accs[g]
            pooled[r, pl.ds(H + g * LANE, LANE)] = zero
        cps = nxt

    pltpu.sync_copy(pooled, out_hbm.at[pl.ds(base, B_PER_W)])


def _make_pool():
    mesh = plsc.VectorSubcoreMesh(core_axis_name="c", subcore_axis_name="s",
                                  num_cores=NC, num_subcores=NS)
    return pl.kernel(
        _pool_body,
        out_type=jax.ShapeDtypeStruct((B, D2), jnp.float32),
        mesh=mesh,
        scratch_types=[
            pltpu.VMEM((2 * B_PER_W, CHUNK_A), jnp.int32),
            pltpu.VMEM((B_PER_W,), jnp.int32),
            pltpu.VMEM((CHUNK_A, D2), jnp.float32),
            pltpu.VMEM((CHUNK_A, D2), jnp.float32),
            pltpu.VMEM((CHUNK_A, D2), jnp.float32),
            pltpu.VMEM((CHUNK_A, D2), jnp.float32),
            pltpu.VMEM((B_PER_W, D2), jnp.float32),
            pltpu.SemaphoreType.DMA,
            pltpu.SemaphoreType.DMA,
        ],
        compiler_params=pltpu.CompilerParams(use_tc_tiling_on_sc=False,
                                             needs_layout_passes=False),
    )


BO = 2048


def _mm_body(x_ref, wt_ref, b_ref, o_ref):
    x = jnp.maximum(x_ref[...][:, :H] * (1.0 / L), 0.0)
    bcol = b_ref[...].reshape(BO, 1)
    o_ref[...] = lax.dot_general(
        wt_ref[...], x, (((0,), (1,)), ((), ())),
        preferred_element_type=jnp.float32) + bcol


def _make_mm():
    grid = (pl.cdiv(OUT, BO),)
    return pl.pallas_call(
        _mm_body,
        grid=grid,
        in_specs=[
            pl.BlockSpec((B, D2), lambda i: (0, 0)),
            pl.BlockSpec((H, BO), lambda i: (0, i)),
            pl.BlockSpec((1, BO), lambda i: (0, i)),
        ],
        out_specs=pl.BlockSpec((BO, B), lambda i: (i, 0)),
        out_shape=jax.ShapeDtypeStruct((OUT, B), jnp.float32),
    )


@jax.jit
def kernel(input_ids, token_type_ids, attention_mask, emb, W, b):
    ids32 = input_ids.astype(jnp.int32)
    key = ((ids32 & 1) << 19) | (ids32 >> 1)
    key = jnp.sort(key, axis=1)
    idsh = key & ((1 << 19) - 1)
    nev = jnp.sum((key >> 19) == 0, axis=1).astype(jnp.int32)
    pad = (jnp.arange(B, dtype=jnp.int32)[:, None] * 61
           + jnp.arange(CHUNK_A - CHUNK_B, dtype=jnp.int32)[None, :] * 977
           ) % VROWS
    ids2 = jnp.concatenate(
        [idsh[:, :CHUNK_A], idsh[:, CHUNK_A:], pad], axis=1)
    ids2 = ids2.reshape(2 * B, CHUNK_A)
    emb2 = emb.reshape(VROWS, D2)
    pooled = _make_pool()(ids2, nev, emb2)
    logits_t = _make_mm()(pooled, W.T, b.reshape(1, OUT))
    return logits_t.T


# speedup vs baseline: 6.1862x; 1.6505x over previous
import jax
import jax.numpy as jnp
from jax import lax
from jax.experimental import pallas as pl
from jax.experimental.pallas import tpu as pltpu
from jax.experimental.pallas import tpu_sc as plsc

B = 1024
L = 200
H = 64
OUT = 100000
NE = 1000000

NC = 2
NS = 16
NW = NC * NS
LANE = 16
B_PER_W = B // NW
NG = H // LANE
D2 = 2 * H

BK = 8192
HB = BK // 2
NBLK = (NE + BK - 1) // BK
VROWS = NBLK * HB

CHUNK_A = 128
CHUNK_B = L - CHUNK_A


def _prep_body(xt_ref, o_ref):
    t = xt_ref[...].T
    o_ref[:, :H] = t[:HB]
    o_ref[:, H:] = t[HB:]


def _make_prep():
    return pl.pallas_call(
        _prep_body,
        grid=(NBLK,),
        in_specs=[pl.BlockSpec((H, BK), lambda i: (0, i))],
        out_specs=pl.BlockSpec((HB, D2), lambda i: (i, 0)),
        out_shape=jax.ShapeDtypeStruct((VROWS, D2), jnp.float32),
    )


def _pool_body(ids2_hbm, nev_hbm, emb2_hbm, out_hbm, idx_all, nev_v,
               rows_a0, rows_b0, rows_a1, rows_b1, pooled, sem0, sem1):
    wid = lax.axis_index("s") * NC + lax.axis_index("c")
    base = wid * B_PER_W
    base2 = pl.multiple_of(base * 2, 8)
    pltpu.sync_copy(ids2_hbm.at[pl.ds(base2, 2 * B_PER_W)], idx_all)
    pltpu.sync_copy(nev_hbm.at[pl.ds(base, B_PER_W)], nev_v)

    zero = jnp.zeros((LANE,), jnp.float32)
    lanes = lax.iota(jnp.int32, LANE)
    bufs = ((rows_a0, rows_b0), (rows_a1, rows_b1))
    sems = (sem0, sem1)

    def start(r):
        ba, bb = bufs[r & 1]
        sem = sems[r & 1]
        c1 = pltpu.async_copy(emb2_hbm.at[idx_all.at[2 * r]], ba, sem)
        c2 = pltpu.async_copy(emb2_hbm.at[idx_all.at[2 * r + 1]], bb, sem)
        return (c1, c2)

    def accum(buf, lo, hi, off, accs):
        def tok(t, accs):
            return tuple(accs[g] + buf[t, pl.ds(off + g * LANE, LANE)]
                         for g in range(NG))

        return lax.fori_loop(lo, hi, tok, accs)

    cps = start(0)
    for r in range(B_PER_W):
        nxt = start(r + 1) if r + 1 < B_PER_W else None
        for c in cps:
            c.wait()
        rows_a, rows_b = bufs[r & 1]
        nv = nev_v[pl.ds((r // LANE) * LANE, LANE)]
        ne = jnp.max(jnp.where(lanes == (r % LANE), nv, jnp.int32(-1)))
        ne_a = jnp.minimum(ne, CHUNK_A)
        ne_b = jnp.maximum(ne - CHUNK_A, 0)
        accs = tuple(jnp.zeros((LANE,), jnp.float32) for _ in range(NG))
        accs = accum(rows_a, 0, ne_a, 0, accs)
        accs = accum(rows_a, ne_a, CHUNK_A, H, accs)
        accs = accum(rows_b, 0, ne_b, 0, accs)
        accs = accum(rows_b, ne_b, CHUNK_B, H, accs)
        for g in range(NG):
            pooled[r, pl.ds(g * LANE, LANE)] = accs[g]
            pooled[r, pl.ds(H + g * LANE, LANE)] = zero
        cps = nxt

    pltpu.sync_copy(pooled, out_hbm.at[pl.ds(base, B_PER_W)])


def _make_pool():
    mesh = plsc.VectorSubcoreMesh(core_axis_name="c", subcore_axis_name="s",
                                  num_cores=NC, num_subcores=NS)
    return pl.kernel(
        _pool_body,
        out_type=jax.ShapeDtypeStruct((B, D2), jnp.float32),
        mesh=mesh,
        scratch_types=[
            pltpu.VMEM((2 * B_PER_W, CHUNK_A), jnp.int32),
            pltpu.VMEM((B_PER_W,), jnp.int32),
            pltpu.VMEM((CHUNK_A, D2), jnp.float32),
            pltpu.VMEM((CHUNK_A, D2), jnp.float32),
            pltpu.VMEM((CHUNK_A, D2), jnp.float32),
            pltpu.VMEM((CHUNK_A, D2), jnp.float32),
            pltpu.VMEM((B_PER_W, D2), jnp.float32),
            pltpu.SemaphoreType.DMA,
            pltpu.SemaphoreType.DMA,
        ],
        compiler_params=pltpu.CompilerParams(needs_layout_passes=False),
    )


BO = 2048


def _mm_body(x_ref, wt_ref, b_ref, o_ref):
    x = jnp.maximum(x_ref[...][:, :H] * (1.0 / L), 0.0)
    bcol = b_ref[...].reshape(BO, 1)
    o_ref[...] = lax.dot_general(
        wt_ref[...], x, (((0,), (1,)), ((), ())),
        preferred_element_type=jnp.float32) + bcol


def _make_mm():
    grid = (pl.cdiv(OUT, BO),)
    return pl.pallas_call(
        _mm_body,
        grid=grid,
        in_specs=[
            pl.BlockSpec((B, D2), lambda i: (0, 0)),
            pl.BlockSpec((H, BO), lambda i: (0, i)),
            pl.BlockSpec((1, BO), lambda i: (0, i)),
        ],
        out_specs=pl.BlockSpec((BO, B), lambda i: (i, 0)),
        out_shape=jax.ShapeDtypeStruct((OUT, B), jnp.float32),
    )


@jax.jit
def kernel(input_ids, token_type_ids, attention_mask, emb, W, b):
    ids32 = input_ids.astype(jnp.int32)
    p = ids32 & (BK - 1)
    half = p >> 12
    row = ((ids32 >> 13) << 12) | (p & (HB - 1))
    key = (half << 19) | row
    key = jnp.sort(key, axis=1)
    idsh = key & ((1 << 19) - 1)
    nev = jnp.sum((key >> 19) == 0, axis=1).astype(jnp.int32)
    pad = (jnp.arange(B, dtype=jnp.int32)[:, None] * 61
           + jnp.arange(CHUNK_A - CHUNK_B, dtype=jnp.int32)[None, :] * 977
           ) % VROWS
    ids2 = jnp.concatenate([idsh[:, :CHUNK_A], idsh[:, CHUNK_A:], pad],
                           axis=1).reshape(2 * B, CHUNK_A)
    emb2 = _make_prep()(emb.T)
    pooled = _make_pool()(ids2, nev, emb2)
    logits_t = _make_mm()(pooled, W.T, b.reshape(1, OUT))
    return logits_t.T


# speedup vs baseline: 6.6071x; 1.0680x over previous
import jax
import jax.numpy as jnp
from jax import lax
from jax.experimental import pallas as pl
from jax.experimental.pallas import tpu as pltpu
from jax.experimental.pallas import tpu_sc as plsc

B = 1024
L = 200
H = 64
OUT = 100000
NE = 1000000

NC = 2
NS = 16
NW = NC * NS
LANE = 16
B_PER_W = B // NW
NG = H // LANE
D2 = 2 * H

BK = 16384
HB = BK // 2
NBLK = (NE + BK - 1) // BK
VROWS = NBLK * HB

CHUNK_A = 128
CHUNK_B = L - CHUNK_A


def _prep_body(xt_ref, o_ref):
    t = xt_ref[...].T
    o_ref[:, :H] = t[:HB]
    o_ref[:, H:] = t[HB:]


def _make_prep():
    return pl.pallas_call(
        _prep_body,
        grid=(NBLK,),
        in_specs=[pl.BlockSpec((H, BK), lambda i: (0, i))],
        out_specs=pl.BlockSpec((HB, D2), lambda i: (i, 0)),
        out_shape=jax.ShapeDtypeStruct((VROWS, D2), jnp.float32),
    )


def _pool_body(ids2_hbm, nev_hbm, emb2_hbm, out_hbm, idx_all, nev_v,
               rows_a0, rows_b0, rows_a1, rows_b1, pooled, sem0, sem1):
    wid = lax.axis_index("s") * NC + lax.axis_index("c")
    base = wid * B_PER_W
    base2 = pl.multiple_of(base * 2, 8)
    pltpu.sync_copy(ids2_hbm.at[pl.ds(base2, 2 * B_PER_W)], idx_all)
    pltpu.sync_copy(nev_hbm.at[pl.ds(base, B_PER_W)], nev_v)

    zero = jnp.zeros((LANE,), jnp.float32)
    lanes = lax.iota(jnp.int32, LANE)
    bufs = ((rows_a0, rows_b0), (rows_a1, rows_b1))
    sems = (sem0, sem1)

    def start(r):
        ba, bb = bufs[r & 1]
        sem = sems[r & 1]
        c1 = pltpu.async_copy(emb2_hbm.at[idx_all.at[2 * r]], ba, sem)
        c2 = pltpu.async_copy(emb2_hbm.at[idx_all.at[2 * r + 1]], bb, sem)
        return (c1, c2)

    def accum(buf, lo, hi, off, accs):
        def tok(t, accs):
            return tuple(accs[g] + buf[t, pl.ds(off + g * LANE, LANE)]
                         for g in range(NG))

        return lax.fori_loop(lo, hi, tok, accs)

    cps = start(0)
    for r in range(B_PER_W):
        nxt = start(r + 1) if r + 1 < B_PER_W else None
        for c in cps:
            c.wait()
        rows_a, rows_b = bufs[r & 1]
        nv = nev_v[pl.ds((r // LANE) * LANE, LANE)]
        ne = jnp.max(jnp.where(lanes == (r % LANE), nv, jnp.int32(-1)))
        ne_a = jnp.minimum(ne, CHUNK_A)
        ne_b = jnp.maximum(ne - CHUNK_A, 0)
        accs = tuple(jnp.zeros((LANE,), jnp.float32) for _ in range(NG))
        accs = accum(rows_a, 0, ne_a, 0, accs)
        accs = accum(rows_a, ne_a, CHUNK_A, H, accs)
        accs = accum(rows_b, 0, ne_b, 0, accs)
        accs = accum(rows_b, ne_b, CHUNK_B, H, accs)
        for g in range(NG):
            pooled[r, pl.ds(g * LANE, LANE)] = accs[g]
            pooled[r, pl.ds(H + g * LANE, LANE)] = zero
        cps = nxt

    pltpu.sync_copy(pooled, out_hbm.at[pl.ds(base, B_PER_W)])


def _make_pool():
    mesh = plsc.VectorSubcoreMesh(core_axis_name="c", subcore_axis_name="s",
                                  num_cores=NC, num_subcores=NS)
    return pl.kernel(
        _pool_body,
        out_type=jax.ShapeDtypeStruct((B, D2), jnp.float32),
        mesh=mesh,
        scratch_types=[
            pltpu.VMEM((2 * B_PER_W, CHUNK_A), jnp.int32),
            pltpu.VMEM((B_PER_W,), jnp.int32),
            pltpu.VMEM((CHUNK_A, D2), jnp.float32),
            pltpu.VMEM((CHUNK_A, D2), jnp.float32),
            pltpu.VMEM((CHUNK_A, D2), jnp.float32),
            pltpu.VMEM((CHUNK_A, D2), jnp.float32),
            pltpu.VMEM((B_PER_W, D2), jnp.float32),
            pltpu.SemaphoreType.DMA,
            pltpu.SemaphoreType.DMA,
        ],
        compiler_params=pltpu.CompilerParams(needs_layout_passes=False),
    )


BO = 2048


def _mm_body(x_ref, wt_ref, b_ref, o_ref):
    x = jnp.maximum(x_ref[...][:, :H] * (1.0 / L), 0.0)
    bcol = b_ref[...].reshape(BO, 1)
    o_ref[...] = lax.dot_general(
        wt_ref[...], x, (((0,), (1,)), ((), ())),
        preferred_element_type=jnp.float32) + bcol


def _make_mm():
    grid = (pl.cdiv(OUT, BO),)
    return pl.pallas_call(
        _mm_body,
        grid=grid,
        in_specs=[
            pl.BlockSpec((B, D2), lambda i: (0, 0)),
            pl.BlockSpec((H, BO), lambda i: (0, i)),
            pl.BlockSpec((1, BO), lambda i: (0, i)),
        ],
        out_specs=pl.BlockSpec((BO, B), lambda i: (i, 0)),
        out_shape=jax.ShapeDtypeStruct((OUT, B), jnp.float32),
    )


@jax.jit
def kernel(input_ids, token_type_ids, attention_mask, emb, W, b):
    ids32 = input_ids.astype(jnp.int32)
    p = ids32 & (BK - 1)
    half = p >> (HB.bit_length() - 1)
    row = ((ids32 >> BK.bit_length() - 1) * HB) | (p & (HB - 1))
    key = (half << 19) | row
    key = jnp.sort(key, axis=1)
    idsh = key & ((1 << 19) - 1)
    nev = jnp.sum((key >> 19) == 0, axis=1).astype(jnp.int32)
    pad = (jnp.arange(B, dtype=jnp.int32)[:, None] * 61
           + jnp.arange(CHUNK_A - CHUNK_B, dtype=jnp.int32)[None, :] * 977
           ) % VROWS
    ids2 = jnp.concatenate([idsh[:, :CHUNK_A], idsh[:, CHUNK_A:], pad],
                           axis=1).reshape(2 * B, CHUNK_A)
    emb2 = _make_prep()(emb.T)
    pooled = _make_pool()(ids2, nev, emb2)
    logits_t = _make_mm()(pooled, W.T, b.reshape(1, OUT))
    return logits_t.T
